# baseline (speedup 1.0000x reference)
import functools

import jax
import jax.numpy as jnp
from jax import lax
from jax.experimental import pallas as pl
from jax.experimental.pallas import tpu as pltpu
from jax.experimental.pallas import tpu_sc as plsc

EMBED = 128
G = 128
NBUF = 4
BLK = 8 * G
SUPER = 2 * BLK


def _sc_gather(idx2d, table):
    n_rows, g = idx2d.shape
    B = n_rows * g
    info = plsc.get_sparse_core_info()
    nw = info.num_cores * info.num_subcores
    b_per_w = B // nw
    n_super = b_per_w // SUPER
    mesh = plsc.VectorSubcoreMesh(core_axis_name="c", subcore_axis_name="s")

    @functools.partial(
        pl.kernel,
        mesh=mesh,
        out_type=jax.ShapeDtypeStruct((B, EMBED), jnp.float32),
        scratch_types=[
            pltpu.VMEM((BLK // G, G), jnp.int32),
            pltpu.VMEM((BLK // G, G), jnp.int32),
            pltpu.VMEM((G, EMBED), jnp.float32),
            pltpu.VMEM((G, EMBED), jnp.float32),
            pltpu.VMEM((G, EMBED), jnp.float32),
            pltpu.VMEM((G, EMBED), jnp.float32),
            pltpu.VMEM_SHARED((1001, EMBED), jnp.float32),
            pltpu.SemaphoreType.DMA,
            pltpu.SemaphoreType.DMA,
            pltpu.SemaphoreType.DMA,
            pltpu.SemaphoreType.DMA,
            pltpu.SemaphoreType.DMA,
            pltpu.SemaphoreType.DMA,
            pltpu.SemaphoreType.DMA,
            pltpu.SemaphoreType.DMA,
            pltpu.SemaphoreType.DMA,
            pltpu.SemaphoreType.DMA,
        ],
    )
    def k(idx_hbm, table_hbm, out_hbm, idx0, idx1, r0, r1, r2, r3, table_sh,
          g0, g1, g2, g3, s0, s1, s2, s3, i0, i1):
        sid = lax.axis_index("s")
        wid = sid * info.num_cores + lax.axis_index("c")
        base = wid * b_per_w
        rows = (r0, r1, r2, r3)
        idxs = (idx0, idx1)
        gsem = (g0, g1, g2, g3)
        ssem = (s0, s1, s2, s3)
        isem = (i0, i1)

        @pl.when(sid == 0)
        def _():
            pltpu.sync_copy(table_hbm, table_sh)

        plsc.subcore_barrier()

        def idx_src(block):
            off = pl.multiple_of(block * (BLK // G), BLK // G)
            return idx_hbm.at[pl.ds(off, BLK // G)]

        def fire_idx(block, par):
            return pltpu.async_copy(idx_src(block), idxs[par], isem[par])

        def idx_wait(par):
            pltpu.make_async_copy(idx_src(0), idxs[par], isem[par]).wait()

        def fire_gather(u, ipar):
            src = table_hbm if u % NBUF == 3 else table_sh
            return pltpu.async_copy(
                src.at[idxs[ipar].at[u % (BLK // G)]],
                rows[u % NBUF],
                gsem[u % NBUF],
            )

        def fire_store(u, blk):
            dst = out_hbm.at[pl.ds(pl.multiple_of(blk + u * G, G), G)]
            return pltpu.async_copy(rows[u % NBUF], dst, ssem[u % NBUF])

        def store_wait(par):
            pltpu.make_async_copy(
                rows[par], out_hbm.at[pl.ds(base, G)], ssem[par]
            ).wait()

        wblock0 = wid * (b_per_w // BLK)
        fire_idx(wblock0, 0)
        fire_idx(wblock0 + 1, 1)

        NU = SUPER // G
        UPB = BLK // G

        def body(i, carry):
            blk = pl.multiple_of(base + i * SUPER, SUPER)
            gh = {}
            sh = {}
            for u in range(NU):
                ipar = u // UPB
                if u == 0:
                    idx_wait(0)
                if u == UPB:
                    idx_wait(1)
                if u >= NBUF:
                    sh[u - NBUF].wait()
                else:
                    @pl.when(i > 0)
                    def _(par=u % NBUF):
                        store_wait(par)

                gh[u] = fire_gather(u, ipar)
                if u >= 1:
                    gh[u - 1].wait()
                    sh[u - 1] = fire_store(u - 1, blk)
                if u == UPB:
                    @pl.when(i + 1 < n_super)
                    def _():
                        fire_idx(wblock0 + 2 * (i + 1), 0)
            gh[NU - 1].wait()
            sh[NU - 1] = fire_store(NU - 1, blk)

            @pl.when(i + 1 < n_super)
            def _():
                fire_idx(wblock0 + 2 * (i + 1) + 1, 1)

            return carry

        lax.fori_loop(0, n_super, body, 0)
        for par in range(NBUF):
            store_wait(par)

    return k(idx2d, table)


def kernel(t, pos_embedding):
    b, h = t.shape
    idx2d = t.astype(jnp.int32).reshape(b * h // G, G)
    out = _sc_gather(idx2d, pos_embedding)
    return out.reshape(b, h, EMBED)


# speedup vs baseline: 1.3569x; 1.3569x over previous
import functools

import jax
import jax.numpy as jnp
from jax import lax
from jax.experimental import pallas as pl
from jax.experimental.pallas import tpu as pltpu
from jax.experimental.pallas import tpu_sc as plsc

EMBED = 128
G = 128
NBUF = 4
BLK = 8 * G
SUPER = 2 * BLK


def _sc_gather(idx2d, table):
    n_rows, g = idx2d.shape
    B = n_rows * g
    info = plsc.get_sparse_core_info()
    nw = info.num_cores * info.num_subcores
    b_per_w = B // nw
    n_super = b_per_w // SUPER
    mesh = plsc.VectorSubcoreMesh(core_axis_name="c", subcore_axis_name="s")

    @functools.partial(
        pl.kernel,
        mesh=mesh,
        out_type=jax.ShapeDtypeStruct((B, EMBED), jnp.float32),
        scratch_types=[
            pltpu.VMEM((BLK // G, G), jnp.int32),
            pltpu.VMEM((BLK // G, G), jnp.int32),
            pltpu.VMEM((G, EMBED), jnp.float32),
            pltpu.VMEM((G, EMBED), jnp.float32),
            pltpu.VMEM((G, EMBED), jnp.float32),
            pltpu.VMEM((G, EMBED), jnp.float32),
            pltpu.VMEM_SHARED((1001, EMBED), jnp.float32),
            pltpu.SemaphoreType.DMA,
            pltpu.SemaphoreType.DMA,
            pltpu.SemaphoreType.DMA,
            pltpu.SemaphoreType.DMA,
            pltpu.SemaphoreType.DMA,
            pltpu.SemaphoreType.DMA,
            pltpu.SemaphoreType.DMA,
            pltpu.SemaphoreType.DMA,
            pltpu.SemaphoreType.DMA,
            pltpu.SemaphoreType.DMA,
        ],
    )
    def k(idx_hbm, table_hbm, out_hbm, idx0, idx1, r0, r1, r2, r3, table_sh,
          g0, g1, g2, g3, s0, s1, s2, s3, i0, i1):
        sid = lax.axis_index("s")
        wid = sid * info.num_cores + lax.axis_index("c")
        base = wid * b_per_w
        rows = (r0, r1, r2, r3)
        idxs = (idx0, idx1)
        gsem = (g0, g1, g2, g3)
        ssem = (s0, s1, s2, s3)
        isem = (i0, i1)

        @pl.when(sid == 0)
        def _():
            pltpu.sync_copy(table_hbm, table_sh)

        plsc.subcore_barrier()

        def idx_src(block):
            off = pl.multiple_of(block * (BLK // G), BLK // G)
            return idx_hbm.at[pl.ds(off, BLK // G)]

        def fire_idx(block, par):
            return pltpu.async_copy(idx_src(block), idxs[par], isem[par])

        def idx_wait(par):
            pltpu.make_async_copy(idx_src(0), idxs[par], isem[par]).wait()

        def fire_gather(u, ipar):
            src = table_sh
            return pltpu.async_copy(
                src.at[idxs[ipar].at[u % (BLK // G)]],
                rows[u % NBUF],
                gsem[u % NBUF],
            )

        def fire_store(u, blk):
            dst = out_hbm.at[pl.ds(pl.multiple_of(blk + u * G, G), G)]
            return pltpu.async_copy(rows[u % NBUF], dst, ssem[u % NBUF])

        def store_wait(par):
            pltpu.make_async_copy(
                rows[par], out_hbm.at[pl.ds(base, G)], ssem[par]
            ).wait()

        wblock0 = wid * (b_per_w // BLK)
        fire_idx(wblock0, 0)
        fire_idx(wblock0 + 1, 1)

        NU = SUPER // G
        UPB = BLK // G

        def body(i, carry):
            blk = pl.multiple_of(base + i * SUPER, SUPER)
            gh = {}
            sh = {}
            for u in range(NU):
                ipar = u // UPB
                if u == 0:
                    idx_wait(0)
                if u == UPB:
                    idx_wait(1)
                if u >= NBUF:
                    sh[u - NBUF].wait()
                else:
                    @pl.when(i > 0)
                    def _(par=u % NBUF):
                        store_wait(par)

                gh[u] = fire_gather(u, ipar)
                if u >= 1:
                    gh[u - 1].wait()
                    sh[u - 1] = fire_store(u - 1, blk)
                if u == UPB:
                    @pl.when(i + 1 < n_super)
                    def _():
                        fire_idx(wblock0 + 2 * (i + 1), 0)
            gh[NU - 1].wait()
            sh[NU - 1] = fire_store(NU - 1, blk)

            @pl.when(i + 1 < n_super)
            def _():
                fire_idx(wblock0 + 2 * (i + 1) + 1, 1)

            return carry

        lax.fori_loop(0, n_super, body, 0)
        for par in range(NBUF):
            store_wait(par)

    return k(idx2d, table)


def kernel(t, pos_embedding):
    b, h = t.shape
    idx2d = t.astype(jnp.int32).reshape(b * h // G, G)
    out = _sc_gather(idx2d, pos_embedding)
    return out.reshape(b, h, EMBED)
